# baseline (speedup 1.0000x reference)
import functools

import jax
import jax.numpy as jnp
from jax import lax
from jax.experimental import pallas as pl
from jax.experimental.pallas import tpu as pltpu
from jax.experimental.pallas import tpu_sc as plsc

B_, K_, N_, C_ = 2, 8, 56564, 64
NC, NS = 2, 16
NW = NC * NS
GROUP = 128
NPADG = (N_ + GROUP - 1) // GROUP
S_ = 16384
GSC = S_ // GROUP
ITERS = GSC // NW
RSTR = 65
KROWS = (K_ + 1) * 16
NB = 4096
NTC = N_ - S_
NTB = (NTC + NB - 1) // NB


def _issue(pw_hbm, x_hbm, g, pwbuf, xb0, xb1, sem):
    n0 = pl.multiple_of(g * GROUP, GROUP)
    pltpu.async_copy(pw_hbm.at[:, pl.ds(n0, GROUP)], pwbuf, sem)
    pltpu.async_copy(x_hbm.at[0, :, pl.ds(n0, GROUP)], xb0, sem)
    pltpu.async_copy(x_hbm.at[1, :, pl.ds(n0, GROUP)], xb1, sem)


def _wait(pw_hbm, x_hbm, pwbuf, xb0, xb1, sem):
    pltpu.make_async_copy(pw_hbm.at[:, pl.ds(0, GROUP)], pwbuf, sem).wait()
    pltpu.make_async_copy(x_hbm.at[0, :, pl.ds(0, GROUP)], xb0, sem).wait()
    pltpu.make_async_copy(x_hbm.at[1, :, pl.ds(0, GROUP)], xb1, sem).wait()


def _sc_body(pw_hbm, x_hbm, out_hbm,
             pwbuf0, xb00, xb10, pwbuf1, xb01, xb11,
             acw0, acw1, acf0, acf1, sem0, sem1):
    cid = lax.axis_index("c")
    sid = lax.axis_index("s")
    wid = sid * NC + cid

    bufs = ((pwbuf0, xb00, xb10, sem0), (pwbuf1, xb01, xb11, sem1))

    zero16 = jnp.zeros((16,), jnp.float32)
    iota16 = lax.iota(jnp.int32, 16)
    iota_r = iota16 * RSTR

    def zero_blk(r, carry):
        for c4 in range(4):
            acw0[pl.ds(r * RSTR + c4 * 16, 16)] = zero16
            acw1[pl.ds(r * RSTR + c4 * 16, 16)] = zero16
        return carry

    lax.fori_loop(0, KROWS, zero_blk, 0)

    def compute(pwbuf, xb0, xb1):
        def sub_body(j, sc):
            j16 = j * 16
            vs = [pwbuf[k, pl.ds(j16, 16)] for k in range(K_)]
            m01 = jnp.maximum(vs[0], vs[1])
            m23 = jnp.maximum(vs[2], vs[3])
            m45 = jnp.maximum(vs[4], vs[5])
            m67 = jnp.maximum(vs[6], vs[7])
            m = jnp.maximum(jnp.maximum(m01, m23), jnp.maximum(m45, m67))
            ks = jnp.full((16,), K_ - 1, jnp.int32)
            for k in range(K_ - 2, -1, -1):
                ks = jnp.where(vs[k] == m, k, ks)
            addr = ks * (16 * RSTR) + iota_r
            ps = pl.ds(j16, 16)

            @plsc.parallel_loop(0, C_, unroll=4)
            def cloop(c):
                a = addr + c
                v0 = xb0[c, ps]
                v1 = xb1[c, ps]
                plsc.addupdate_scatter(acw0, [a], v0)
                plsc.addupdate_scatter(acw1, [a], v1)

            return sc

        lax.fori_loop(0, K_, sub_body, 0)

    _issue(pw_hbm, x_hbm, wid, *bufs[0])

    def outer(io, carry):
        for par in range(2):
            i = 2 * io + par
            g = i * NW + wid
            pwbuf, xb0, xb1, sem = bufs[par]
            _wait(pw_hbm, x_hbm, pwbuf, xb0, xb1, sem)
            gn = (i + 1) * NW + wid

            @pl.when(gn < GSC)
            def _():
                _issue(pw_hbm, x_hbm, gn, *bufs[1 - par])

            compute(pwbuf, xb0, xb1)
        return carry

    lax.fori_loop(0, ITERS // 2, outer, 0)

    def red_kc(kc, carry):
        k = kc // 4
        co = (kc % 4) * 16
        base = k * (16 * RSTR) + co
        t0 = [acw0[pl.ds(base + l * RSTR, 16)] for l in range(16)]
        t1 = [acw1[pl.ds(base + l * RSTR, 16)] for l in range(16)]
        s0, s1 = t0[0], t1[0]
        for l in range(1, 16):
            s0 = s0 + t0[l]
            s1 = s1 + t1[l]
        acf0[k, pl.ds(co, 16)] = s0
        acf1[k, pl.ds(co, 16)] = s1
        return carry

    lax.fori_loop(0, K_ * 4, red_kc, 0)

    obase = wid * 2 * K_
    pltpu.sync_copy(acf0, out_hbm.at[pl.ds(obase, K_)])
    pltpu.sync_copy(acf1, out_hbm.at[pl.ds(obase + K_, K_)])


@functools.partial(
    pl.kernel,
    out_type=jax.ShapeDtypeStruct((NW * 2 * K_, C_), jnp.float32),
    mesh=plsc.VectorSubcoreMesh(core_axis_name="c", subcore_axis_name="s"),
    compiler_params=pltpu.CompilerParams(
        needs_layout_passes=False, skip_device_barrier=True),
    scratch_types=[
        pltpu.VMEM((K_, GROUP), jnp.float32),
        pltpu.VMEM((C_, GROUP), jnp.float32),
        pltpu.VMEM((C_, GROUP), jnp.float32),
        pltpu.VMEM((K_, GROUP), jnp.float32),
        pltpu.VMEM((C_, GROUP), jnp.float32),
        pltpu.VMEM((C_, GROUP), jnp.float32),
        pltpu.VMEM((KROWS * RSTR,), jnp.float32),
        pltpu.VMEM((KROWS * RSTR,), jnp.float32),
        pltpu.VMEM((K_, C_), jnp.float32),
        pltpu.VMEM((K_, C_), jnp.float32),
        pltpu.SemaphoreType.DMA,
        pltpu.SemaphoreType.DMA,
    ],
)
def _selection_sc(pw_hbm, x_hbm, out_hbm, *scratch):
    _sc_body(pw_hbm, x_hbm, out_hbm, *scratch)


def _tc_body(pw_ref, x_ref, o_ref):
    i = pl.program_id(0)
    pwb = pw_ref[...]
    m = jnp.max(pwb, axis=0, keepdims=True)
    kidx = lax.broadcasted_iota(jnp.int32, (K_, NB), 0)
    first = jnp.min(jnp.where(pwb == m, kidx, K_), axis=0, keepdims=True)
    gcol = i * NB + lax.broadcasted_iota(jnp.int32, (1, NB), 1)
    oh = jnp.where((kidx == first) & (gcol < NTC), 1.0, 0.0).astype(jnp.float32)
    dn = (((1,), (1,)), ((), ()))
    p0 = lax.dot_general(oh, x_ref[0], dn, preferred_element_type=jnp.float32)
    p1 = lax.dot_general(oh, x_ref[1], dn, preferred_element_type=jnp.float32)

    @pl.when(i == 0)
    def _():
        o_ref[0] = p0
        o_ref[1] = p1

    @pl.when(i > 0)
    def _():
        o_ref[0] += p0
        o_ref[1] += p1


_selection_tc = pl.pallas_call(
    _tc_body,
    grid=(NTB,),
    in_specs=[
        pl.BlockSpec((K_, NB), lambda i: (0, i)),
        pl.BlockSpec((B_, C_, NB), lambda i: (0, 0, S_ // NB + i)),
    ],
    out_specs=pl.BlockSpec((B_, K_, C_), lambda i: (0, 0, 0)),
    out_shape=jax.ShapeDtypeStruct((B_, K_, C_), jnp.float32),
)


def kernel(x, point_weight, tau):
    del tau
    pw_sc = lax.slice(point_weight, (0, 0, 0, 0),
                      (1, K_, S_, 1)).reshape(K_, S_)
    pw_tc = lax.slice(point_weight, (0, 0, S_, 0),
                      (1, K_, N_, 1)).reshape(K_, N_ - S_)
    xt = x.transpose(0, 2, 1)
    partial = _selection_sc(pw_sc, xt)
    tc_out = _selection_tc(pw_tc, xt)
    return partial.reshape(NW, B_, K_, C_).sum(axis=0) + tc_out


# speedup vs baseline: 1.0377x; 1.0377x over previous
import functools

import jax
import jax.numpy as jnp
from jax import lax
from jax.experimental import pallas as pl
from jax.experimental.pallas import tpu as pltpu
from jax.experimental.pallas import tpu_sc as plsc

B_, K_, N_, C_ = 2, 8, 56564, 64
NC, NS = 2, 16
NW = NC * NS
GROUP = 128
NPADG = (N_ + GROUP - 1) // GROUP
S_ = 16384
GSC = S_ // GROUP
ITERS = GSC // NW
RSTR = 65
KROWS = (K_ + 1) * 16
NB = 8192
NTC = N_ - S_
NTB = (NTC + NB - 1) // NB


def _issue(pw_hbm, x_hbm, g, pwbuf, xb0, xb1, sem):
    n0 = pl.multiple_of(g * GROUP, GROUP)
    pltpu.async_copy(pw_hbm.at[:, pl.ds(n0, GROUP)], pwbuf, sem)
    pltpu.async_copy(x_hbm.at[0, :, pl.ds(n0, GROUP)], xb0, sem)
    pltpu.async_copy(x_hbm.at[1, :, pl.ds(n0, GROUP)], xb1, sem)


def _wait(pw_hbm, x_hbm, pwbuf, xb0, xb1, sem):
    pltpu.make_async_copy(pw_hbm.at[:, pl.ds(0, GROUP)], pwbuf, sem).wait()
    pltpu.make_async_copy(x_hbm.at[0, :, pl.ds(0, GROUP)], xb0, sem).wait()
    pltpu.make_async_copy(x_hbm.at[1, :, pl.ds(0, GROUP)], xb1, sem).wait()


def _sc_body(pw_hbm, x_hbm, out_hbm,
             pwbuf0, xb00, xb10, pwbuf1, xb01, xb11,
             acw0, acw1, acf, idxv, shared, sem0, sem1):
    cid = lax.axis_index("c")
    sid = lax.axis_index("s")
    wid = sid * NC + cid

    bufs = ((pwbuf0, xb00, xb10, sem0), (pwbuf1, xb01, xb11, sem1))

    zero16 = jnp.zeros((16,), jnp.float32)
    iota16 = lax.iota(jnp.int32, 16)
    iota_r = iota16 * RSTR
    idxv[...] = iota16

    def zero_acf(r, carry):
        for c4 in range(4):
            acf[r, pl.ds(c4 * 16, 16)] = zero16
        return carry

    lax.fori_loop(0, 2 * K_, zero_acf, 0)

    @pl.when(sid == 0)
    def _():
        pltpu.sync_copy(acf, shared)

    plsc.subcore_barrier()

    def zero_blk(r, carry):
        for c4 in range(4):
            acw0[pl.ds(r * RSTR + c4 * 16, 16)] = zero16
            acw1[pl.ds(r * RSTR + c4 * 16, 16)] = zero16
        return carry

    lax.fori_loop(0, KROWS, zero_blk, 0)

    def compute(pwbuf, xb0, xb1):
        def sub_body(j, sc):
            j16 = j * 16
            vs = [pwbuf[k, pl.ds(j16, 16)] for k in range(K_)]
            m01 = jnp.maximum(vs[0], vs[1])
            m23 = jnp.maximum(vs[2], vs[3])
            m45 = jnp.maximum(vs[4], vs[5])
            m67 = jnp.maximum(vs[6], vs[7])
            m = jnp.maximum(jnp.maximum(m01, m23), jnp.maximum(m45, m67))
            ks = jnp.full((16,), K_ - 1, jnp.int32)
            for k in range(K_ - 2, -1, -1):
                ks = jnp.where(vs[k] == m, k, ks)
            addr = ks * (16 * RSTR) + iota_r
            ps = pl.ds(j16, 16)

            @plsc.parallel_loop(0, C_, unroll=4)
            def cloop(c):
                a = addr + c
                v0 = xb0[c, ps]
                v1 = xb1[c, ps]
                plsc.addupdate_scatter(acw0, [a], v0)
                plsc.addupdate_scatter(acw1, [a], v1)

            return sc

        lax.fori_loop(0, K_, sub_body, 0)

    _issue(pw_hbm, x_hbm, wid, *bufs[0])

    def outer(io, carry):
        for par in range(2):
            i = 2 * io + par
            g = i * NW + wid
            pwbuf, xb0, xb1, sem = bufs[par]
            _wait(pw_hbm, x_hbm, pwbuf, xb0, xb1, sem)
            gn = (i + 1) * NW + wid

            @pl.when(gn < GSC)
            def _():
                _issue(pw_hbm, x_hbm, gn, *bufs[1 - par])

            compute(pwbuf, xb0, xb1)
        return carry

    lax.fori_loop(0, ITERS // 2, outer, 0)

    def red_kc(kc, carry):
        k = kc // 4
        co = (kc % 4) * 16
        base = k * (16 * RSTR) + co
        t0 = [acw0[pl.ds(base + l * RSTR, 16)] for l in range(16)]
        t1 = [acw1[pl.ds(base + l * RSTR, 16)] for l in range(16)]
        s0, s1 = t0[0], t1[0]
        for l in range(1, 16):
            s0 = s0 + t0[l]
            s1 = s1 + t1[l]
        acf[k, pl.ds(co, 16)] = s0
        acf[K_ + k, pl.ds(co, 16)] = s1
        return carry

    lax.fori_loop(0, K_ * 4, red_kc, 0)

    pltpu.sync_copy(acf, shared.at[idxv], add=True)
    plsc.subcore_barrier()

    @pl.when(sid == 0)
    def _():
        pltpu.sync_copy(shared, out_hbm.at[pl.ds(cid * 2 * K_, 2 * K_)])


@functools.partial(
    pl.kernel,
    out_type=jax.ShapeDtypeStruct((NC * 2 * K_, C_), jnp.float32),
    mesh=plsc.VectorSubcoreMesh(core_axis_name="c", subcore_axis_name="s"),
    compiler_params=pltpu.CompilerParams(
        needs_layout_passes=False, skip_device_barrier=True),
    scratch_types=[
        pltpu.VMEM((K_, GROUP), jnp.float32),
        pltpu.VMEM((C_, GROUP), jnp.float32),
        pltpu.VMEM((C_, GROUP), jnp.float32),
        pltpu.VMEM((K_, GROUP), jnp.float32),
        pltpu.VMEM((C_, GROUP), jnp.float32),
        pltpu.VMEM((C_, GROUP), jnp.float32),
        pltpu.VMEM((KROWS * RSTR,), jnp.float32),
        pltpu.VMEM((KROWS * RSTR,), jnp.float32),
        pltpu.VMEM((2 * K_, C_), jnp.float32),
        pltpu.VMEM((16,), jnp.int32),
        pltpu.VMEM_SHARED((2 * K_, C_), jnp.float32),
        pltpu.SemaphoreType.DMA,
        pltpu.SemaphoreType.DMA,
    ],
)
def _selection_sc(pw_hbm, x_hbm, out_hbm, *scratch):
    _sc_body(pw_hbm, x_hbm, out_hbm, *scratch)


def _tc_body(pw_ref, x_ref, o_ref):
    i = pl.program_id(0)
    pwb = pw_ref[...]
    m = jnp.max(pwb, axis=0, keepdims=True)
    kidx = lax.broadcasted_iota(jnp.int32, (K_, NB), 0)
    first = jnp.min(jnp.where(pwb == m, kidx, K_), axis=0, keepdims=True)
    gcol = i * NB + lax.broadcasted_iota(jnp.int32, (1, NB), 1)
    oh = jnp.where((kidx == first) & (gcol < NTC), 1.0, 0.0).astype(jnp.float32)
    dn = (((1,), (1,)), ((), ()))
    p0 = lax.dot_general(oh, x_ref[0], dn, preferred_element_type=jnp.float32)
    p1 = lax.dot_general(oh, x_ref[1], dn, preferred_element_type=jnp.float32)

    @pl.when(i == 0)
    def _():
        o_ref[0] = p0
        o_ref[1] = p1

    @pl.when(i > 0)
    def _():
        o_ref[0] += p0
        o_ref[1] += p1


_selection_tc = pl.pallas_call(
    _tc_body,
    grid=(NTB,),
    in_specs=[
        pl.BlockSpec((K_, NB), lambda i: (0, i)),
        pl.BlockSpec((B_, C_, NB), lambda i: (0, 0, S_ // NB + i)),
    ],
    out_specs=pl.BlockSpec((B_, K_, C_), lambda i: (0, 0, 0)),
    out_shape=jax.ShapeDtypeStruct((B_, K_, C_), jnp.float32),
)


def kernel(x, point_weight, tau):
    del tau
    pw_sc = lax.slice(point_weight, (0, 0, 0, 0),
                      (1, K_, S_, 1)).reshape(K_, S_)
    pw_tc = lax.slice(point_weight, (0, 0, S_, 0),
                      (1, K_, N_, 1)).reshape(K_, N_ - S_)
    xt = x.transpose(0, 2, 1)
    partial = _selection_sc(pw_sc, xt)
    tc_out = _selection_tc(pw_tc, xt)
    return partial.reshape(NC, B_, K_, C_).sum(axis=0) + tc_out


# speedup vs baseline: 1.0497x; 1.0115x over previous
import functools

import jax
import jax.numpy as jnp
from jax import lax
from jax.experimental import pallas as pl
from jax.experimental.pallas import tpu as pltpu
from jax.experimental.pallas import tpu_sc as plsc

B_, K_, N_, C_ = 2, 8, 56564, 64
NC, NS = 2, 16
NW = NC * NS
GROUP = 128
NPADG = (N_ + GROUP - 1) // GROUP
S_ = 16384
GSC = S_ // GROUP
ITERS = GSC // NW
RSTR = 65
KROWS = (K_ + 1) * 16
NB = 8192
NTC = N_ - S_
NTB = (NTC + NB - 1) // NB


def _issue(pw_hbm, x_hbm, g, pwbuf, xb0, xb1, sem):
    n0 = pl.multiple_of(g * GROUP, GROUP)
    pltpu.async_copy(pw_hbm.at[:, pl.ds(n0, GROUP)], pwbuf, sem)
    pltpu.async_copy(x_hbm.at[0, :, pl.ds(n0, GROUP)], xb0, sem)
    pltpu.async_copy(x_hbm.at[1, :, pl.ds(n0, GROUP)], xb1, sem)


def _wait(pw_hbm, x_hbm, pwbuf, xb0, xb1, sem):
    pltpu.make_async_copy(pw_hbm.at[:, pl.ds(0, GROUP)], pwbuf, sem).wait()
    pltpu.make_async_copy(x_hbm.at[0, :, pl.ds(0, GROUP)], xb0, sem).wait()
    pltpu.make_async_copy(x_hbm.at[1, :, pl.ds(0, GROUP)], xb1, sem).wait()


def _sc_body(pw_hbm, x_hbm, out_hbm,
             pwbuf0, xb00, xb10, pwbuf1, xb01, xb11,
             acw0, acw1, acf, sem0, sem1):
    cid = lax.axis_index("c")
    sid = lax.axis_index("s")
    wid = sid * NC + cid

    bufs = ((pwbuf0, xb00, xb10, sem0), (pwbuf1, xb01, xb11, sem1))

    zero16 = jnp.zeros((16,), jnp.float32)
    iota16 = lax.iota(jnp.int32, 16)
    iota_r = iota16 * RSTR

    def zero_blk(r, carry):
        for c4 in range(4):
            acw0[pl.ds(r * RSTR + c4 * 16, 16)] = zero16
            acw1[pl.ds(r * RSTR + c4 * 16, 16)] = zero16
        return carry

    lax.fori_loop(0, KROWS, zero_blk, 0)

    def compute(pwbuf, xb0, xb1):
        def sub_body(j, sc):
            j16 = j * 16
            vs = [pwbuf[k, pl.ds(j16, 16)] for k in range(K_)]
            m01 = jnp.maximum(vs[0], vs[1])
            m23 = jnp.maximum(vs[2], vs[3])
            m45 = jnp.maximum(vs[4], vs[5])
            m67 = jnp.maximum(vs[6], vs[7])
            m = jnp.maximum(jnp.maximum(m01, m23), jnp.maximum(m45, m67))
            ks = jnp.full((16,), K_ - 1, jnp.int32)
            for k in range(K_ - 2, -1, -1):
                ks = jnp.where(vs[k] == m, k, ks)
            addr = ks * (16 * RSTR) + iota_r
            ps = pl.ds(j16, 16)

            @plsc.parallel_loop(0, C_, unroll=4)
            def cloop(c):
                a = addr + c
                v0 = xb0[c, ps]
                v1 = xb1[c, ps]
                plsc.addupdate_scatter(acw0, [a], v0)
                plsc.addupdate_scatter(acw1, [a], v1)

            return sc

        lax.fori_loop(0, K_, sub_body, 0)

    _issue(pw_hbm, x_hbm, wid, *bufs[0])

    def outer(io, carry):
        for par in range(2):
            i = 2 * io + par
            g = i * NW + wid
            pwbuf, xb0, xb1, sem = bufs[par]
            _wait(pw_hbm, x_hbm, pwbuf, xb0, xb1, sem)
            gn = (i + 1) * NW + wid

            @pl.when(gn < GSC)
            def _():
                _issue(pw_hbm, x_hbm, gn, *bufs[1 - par])

            compute(pwbuf, xb0, xb1)
        return carry

    lax.fori_loop(0, ITERS // 2, outer, 0)

    def red_kc(kc, carry):
        k = kc // 4
        co = (kc % 4) * 16
        base = k * (16 * RSTR) + co
        t0 = [acw0[pl.ds(base + l * RSTR, 16)] for l in range(16)]
        t1 = [acw1[pl.ds(base + l * RSTR, 16)] for l in range(16)]
        s0, s1 = t0[0], t1[0]
        for l in range(1, 16):
            s0 = s0 + t0[l]
            s1 = s1 + t1[l]
        acf[k, pl.ds(co, 16)] = s0
        acf[K_ + k, pl.ds(co, 16)] = s1
        return carry

    lax.fori_loop(0, K_ * 4, red_kc, 0)

    pltpu.sync_copy(acf, out_hbm.at[pl.ds(wid * 2 * K_, 2 * K_)])


@functools.partial(
    pl.kernel,
    out_type=jax.ShapeDtypeStruct((NW * 2 * K_, C_), jnp.float32),
    mesh=plsc.VectorSubcoreMesh(core_axis_name="c", subcore_axis_name="s"),
    compiler_params=pltpu.CompilerParams(
        needs_layout_passes=False, skip_device_barrier=True),
    scratch_types=[
        pltpu.VMEM((K_, GROUP), jnp.float32),
        pltpu.VMEM((C_, GROUP), jnp.float32),
        pltpu.VMEM((C_, GROUP), jnp.float32),
        pltpu.VMEM((K_, GROUP), jnp.float32),
        pltpu.VMEM((C_, GROUP), jnp.float32),
        pltpu.VMEM((C_, GROUP), jnp.float32),
        pltpu.VMEM((KROWS * RSTR,), jnp.float32),
        pltpu.VMEM((KROWS * RSTR,), jnp.float32),
        pltpu.VMEM((2 * K_, C_), jnp.float32),
        pltpu.SemaphoreType.DMA,
        pltpu.SemaphoreType.DMA,
    ],
)
def _selection_sc(pw_hbm, x_hbm, out_hbm, *scratch):
    _sc_body(pw_hbm, x_hbm, out_hbm, *scratch)


def _tc_body(pw_ref, x_ref, o_ref):
    i = pl.program_id(0)
    pwb = pw_ref[...]
    m = jnp.max(pwb, axis=0, keepdims=True)
    kidx = lax.broadcasted_iota(jnp.int32, (K_, NB), 0)
    first = jnp.min(jnp.where(pwb == m, kidx, K_), axis=0, keepdims=True)
    gcol = i * NB + lax.broadcasted_iota(jnp.int32, (1, NB), 1)
    oh = jnp.where((kidx == first) & (gcol < NTC), 1.0, 0.0).astype(jnp.float32)
    dn = (((1,), (1,)), ((), ()))
    p0 = lax.dot_general(oh, x_ref[0], dn, preferred_element_type=jnp.float32)
    p1 = lax.dot_general(oh, x_ref[1], dn, preferred_element_type=jnp.float32)

    @pl.when(i == 0)
    def _():
        o_ref[0] = p0
        o_ref[1] = p1

    @pl.when(i > 0)
    def _():
        o_ref[0] += p0
        o_ref[1] += p1


_selection_tc = pl.pallas_call(
    _tc_body,
    grid=(NTB,),
    in_specs=[
        pl.BlockSpec((K_, NB), lambda i: (0, i)),
        pl.BlockSpec((B_, C_, NB), lambda i: (0, 0, S_ // NB + i)),
    ],
    out_specs=pl.BlockSpec((B_, K_, C_), lambda i: (0, 0, 0)),
    out_shape=jax.ShapeDtypeStruct((B_, K_, C_), jnp.float32),
)


def kernel(x, point_weight, tau):
    del tau
    pw_sc = lax.slice(point_weight, (0, 0, 0, 0),
                      (1, K_, S_, 1)).reshape(K_, S_)
    pw_tc = lax.slice(point_weight, (0, 0, S_, 0),
                      (1, K_, N_, 1)).reshape(K_, N_ - S_)
    xt = x.transpose(0, 2, 1)
    partial = _selection_sc(pw_sc, xt)
    tc_out = _selection_tc(pw_tc, xt)
    allp = jnp.concatenate(
        [partial.reshape(NW, B_, K_, C_), tc_out[None]], axis=0)
    return allp.sum(axis=0)


# speedup vs baseline: 1.0518x; 1.0021x over previous
import functools

import jax
import jax.numpy as jnp
from jax import lax
from jax.experimental import pallas as pl
from jax.experimental.pallas import tpu as pltpu
from jax.experimental.pallas import tpu_sc as plsc

B_, K_, N_, C_ = 2, 8, 56564, 64
NC, NS = 2, 16
NW = NC * NS
GROUP = 128
NPADG = (N_ + GROUP - 1) // GROUP
S_ = 16384
GSC = S_ // GROUP
ITERS = GSC // NW
RSTR = 65
KROWS = (K_ + 1) * 16
NB = 8192
NTC = N_ - S_
NTB = (NTC + NB - 1) // NB


def _issue(pw_hbm, x_hbm, g, pwbuf, xb0, xb1, sem):
    n0 = pl.multiple_of(g * GROUP, GROUP)
    pltpu.async_copy(pw_hbm.at[:, pl.ds(n0, GROUP)], pwbuf, sem)
    pltpu.async_copy(x_hbm.at[0, :, pl.ds(n0, GROUP)], xb0, sem)
    pltpu.async_copy(x_hbm.at[1, :, pl.ds(n0, GROUP)], xb1, sem)


def _wait(pw_hbm, x_hbm, pwbuf, xb0, xb1, sem):
    pltpu.make_async_copy(pw_hbm.at[:, pl.ds(0, GROUP)], pwbuf, sem).wait()
    pltpu.make_async_copy(x_hbm.at[0, :, pl.ds(0, GROUP)], xb0, sem).wait()
    pltpu.make_async_copy(x_hbm.at[1, :, pl.ds(0, GROUP)], xb1, sem).wait()


def _sc_body(pw_hbm, x_hbm, out_hbm,
             pwbuf0, xb00, xb10, pwbuf1, xb01, xb11,
             acw0, acw1, acf, sem0, sem1):
    cid = lax.axis_index("c")
    sid = lax.axis_index("s")
    wid = sid * NC + cid

    bufs = ((pwbuf0, xb00, xb10, sem0), (pwbuf1, xb01, xb11, sem1))

    zero16 = jnp.zeros((16,), jnp.float32)
    iota16 = lax.iota(jnp.int32, 16)
    iota_r = iota16 * RSTR

    def zero_blk(r, carry):
        for c4 in range(4):
            acw0[pl.ds(r * RSTR + c4 * 16, 16)] = zero16
            acw1[pl.ds(r * RSTR + c4 * 16, 16)] = zero16
        return carry

    lax.fori_loop(0, KROWS, zero_blk, 0)

    def compute(pwbuf, xb0, xb1):
        def sub_body(j, sc):
            j16 = j * 16
            vs = [pwbuf[k, pl.ds(j16, 16)] for k in range(K_)]
            m01 = jnp.maximum(vs[0], vs[1])
            m23 = jnp.maximum(vs[2], vs[3])
            m45 = jnp.maximum(vs[4], vs[5])
            m67 = jnp.maximum(vs[6], vs[7])
            m = jnp.maximum(jnp.maximum(m01, m23), jnp.maximum(m45, m67))
            ks = jnp.full((16,), K_ - 1, jnp.int32)
            for k in range(K_ - 2, -1, -1):
                ks = jnp.where(vs[k] == m, k, ks)
            addr = ks * (16 * RSTR) + iota_r
            ps = pl.ds(j16, 16)

            @plsc.parallel_loop(0, C_, unroll=4)
            def cloop(c):
                a = addr + c
                v0 = xb0[c, ps]
                v1 = xb1[c, ps]
                plsc.addupdate_scatter(acw0, [a], v0)
                plsc.addupdate_scatter(acw1, [a], v1)

            return sc

        lax.fori_loop(0, K_, sub_body, 0)

    _issue(pw_hbm, x_hbm, wid, *bufs[0])

    def outer(io, carry):
        for par in range(2):
            i = 2 * io + par
            g = i * NW + wid
            pwbuf, xb0, xb1, sem = bufs[par]
            _wait(pw_hbm, x_hbm, pwbuf, xb0, xb1, sem)
            gn = (i + 1) * NW + wid

            @pl.when(gn < GSC)
            def _():
                _issue(pw_hbm, x_hbm, gn, *bufs[1 - par])

            compute(pwbuf, xb0, xb1)
        return carry

    lax.fori_loop(0, ITERS // 2, outer, 0)

    def red_kc(kc, carry):
        k = kc // 4
        co = (kc % 4) * 16
        base = k * (16 * RSTR) + co
        t0 = [acw0[pl.ds(base + l * RSTR, 16)] for l in range(16)]
        t1 = [acw1[pl.ds(base + l * RSTR, 16)] for l in range(16)]
        s0, s1 = t0[0], t1[0]
        for l in range(1, 16):
            s0 = s0 + t0[l]
            s1 = s1 + t1[l]
        acf[k, pl.ds(co, 16)] = s0
        acf[K_ + k, pl.ds(co, 16)] = s1
        return carry

    lax.fori_loop(0, K_ * 4, red_kc, 0)

    pltpu.sync_copy(acf, out_hbm.at[pl.ds(wid * 2 * K_, 2 * K_)])


@functools.partial(
    pl.kernel,
    out_type=jax.ShapeDtypeStruct((NW * 2 * K_, C_), jnp.float32),
    mesh=plsc.VectorSubcoreMesh(core_axis_name="c", subcore_axis_name="s"),
    compiler_params=pltpu.CompilerParams(
        needs_layout_passes=False, skip_device_barrier=True),
    scratch_types=[
        pltpu.VMEM((K_, GROUP), jnp.float32),
        pltpu.VMEM((C_, GROUP), jnp.float32),
        pltpu.VMEM((C_, GROUP), jnp.float32),
        pltpu.VMEM((K_, GROUP), jnp.float32),
        pltpu.VMEM((C_, GROUP), jnp.float32),
        pltpu.VMEM((C_, GROUP), jnp.float32),
        pltpu.VMEM((KROWS * RSTR,), jnp.float32),
        pltpu.VMEM((KROWS * RSTR,), jnp.float32),
        pltpu.VMEM((2 * K_, C_), jnp.float32),
        pltpu.SemaphoreType.DMA,
        pltpu.SemaphoreType.DMA,
    ],
)
def _selection_sc(pw_hbm, x_hbm, out_hbm, *scratch):
    _sc_body(pw_hbm, x_hbm, out_hbm, *scratch)


def _tc_body(pw_ref, x_ref, o_ref):
    i = pl.program_id(0)
    pwb = pw_ref[...]
    m = jnp.max(pwb, axis=0, keepdims=True)
    kidx = lax.broadcasted_iota(jnp.int32, (K_, NB), 0)
    first = jnp.min(jnp.where(pwb == m, kidx, K_), axis=0, keepdims=True)
    gcol = i * NB + lax.broadcasted_iota(jnp.int32, (1, NB), 1)
    oh = jnp.where((kidx == first) & (gcol < NTC), 1.0, 0.0).astype(jnp.float32)
    dn = (((1,), (1,)), ((), ()))
    xc = x_ref[...].reshape(B_ * C_, NB)
    p = lax.dot_general(oh, xc, dn, preferred_element_type=jnp.float32)
    p = p.reshape(K_, B_, C_).swapaxes(0, 1)

    @pl.when(i == 0)
    def _():
        o_ref[...] = p

    @pl.when(i > 0)
    def _():
        o_ref[...] += p


_selection_tc = pl.pallas_call(
    _tc_body,
    grid=(NTB,),
    in_specs=[
        pl.BlockSpec((K_, NB), lambda i: (0, i)),
        pl.BlockSpec((B_, C_, NB), lambda i: (0, 0, S_ // NB + i)),
    ],
    out_specs=pl.BlockSpec((B_, K_, C_), lambda i: (0, 0, 0)),
    out_shape=jax.ShapeDtypeStruct((B_, K_, C_), jnp.float32),
)


def kernel(x, point_weight, tau):
    del tau
    pw_sc = lax.slice(point_weight, (0, 0, 0, 0),
                      (1, K_, S_, 1)).reshape(K_, S_)
    pw_tc = lax.slice(point_weight, (0, 0, S_, 0),
                      (1, K_, N_, 1)).reshape(K_, N_ - S_)
    xt = x.transpose(0, 2, 1)
    partial = _selection_sc(pw_sc, xt)
    tc_out = _selection_tc(pw_tc, xt)
    allp = jnp.concatenate(
        [partial.reshape(NW, B_, K_, C_), tc_out[None]], axis=0)
    return allp.sum(axis=0)
